# baseline (speedup 1.0000x reference)
import functools

import jax
import jax.numpy as jnp
from jax import lax
from jax.experimental import pallas as pl
from jax.experimental.pallas import tpu as pltpu
from jax.experimental.pallas import tpu_sc as plsc

N = 2048
D = 2048
NUM_CLASSES = 751
KP1 = 6
EPS_GIN = 0.3
ZERO_ROW = N

BR = 256


def _norms_body(x_ref, o_ref):
    x = x_ref[...]
    n = jnp.sqrt(jnp.sum(x * x, axis=1))
    o_ref[...] = jnp.broadcast_to((1.0 / (n + 1e-12))[:, None], o_ref.shape)


def _norms(x):
    return pl.pallas_call(
        _norms_body,
        grid=(1,),
        in_specs=[pl.BlockSpec((N, D), lambda i: (0, 0))],
        out_specs=pl.BlockSpec((N, 8), lambda i: (0, 0)),
        out_shape=jax.ShapeDtypeStruct((N, 8), jnp.float32),
    )(x)


def _topk_body(x_blk_ref, x_full_ref, rn_ref, idx_ref):
    a = x_blk_ref[...]
    rs = jnp.sqrt(jnp.sum(a * a, axis=1, keepdims=True))
    a = a / (rs + 1e-12)
    b = x_full_ref[...] * rn_ref[:, 0:1]
    sim = lax.dot_general(a, b, (((1,), (1,)), ((), ())),
                          preferred_element_type=jnp.float32)
    d = 1.0 - sim
    col = lax.broadcasted_iota(jnp.int32, d.shape, 1)
    for t in range(KP1):
        m = jnp.min(d, axis=1, keepdims=True)
        idx = jnp.min(jnp.where(d == m, col, N), axis=1)
        idx_ref[t, :] = idx
        d = jnp.where(col == idx[:, None], jnp.float32(1e9), d)
    zero = jnp.zeros((idx_ref.shape[1],), jnp.int32)
    for t in range(KP1, 8):
        idx_ref[t, :] = zero


def _topk(x, rn8):
    return pl.pallas_call(
        _topk_body,
        grid=(N // BR,),
        in_specs=[
            pl.BlockSpec((BR, D), lambda i: (i, 0)),
            pl.BlockSpec((N, D), lambda i: (0, 0)),
            pl.BlockSpec((N, 8), lambda i: (0, 0)),
        ],
        out_specs=pl.BlockSpec((8, BR), lambda i: (0, i)),
        out_shape=jax.ShapeDtypeStruct((8, N), jnp.int32),
    )(x, x, rn8)


GN = 4
NG = None


def _sc_neigh_sum(fwd_flat, xpad):
    info = plsc.get_sparse_core_info()
    nc, ns = info.num_cores, info.num_subcores
    nw = nc * ns
    npw = N // nw
    ng = npw // GN
    rows = GN * KP1
    mesh = plsc.VectorSubcoreMesh(core_axis_name="c", subcore_axis_name="s")

    @functools.partial(
        pl.kernel,
        out_type=jax.ShapeDtypeStruct((N, D), jnp.float32),
        mesh=mesh,
        compiler_params=pltpu.CompilerParams(needs_layout_passes=False),
        scratch_types=[
            pltpu.VMEM((KP1 * N,), jnp.int32),
            pltpu.VMEM((ng, rows), jnp.int32),
            pltpu.VMEM((rows, D), jnp.float32),
            pltpu.VMEM((rows, D), jnp.float32),
            pltpu.VMEM((GN, D), jnp.float32),
            pltpu.VMEM((GN, D), jnp.float32),
            pltpu.SemaphoreType.DMA,
            pltpu.SemaphoreType.DMA,
            pltpu.SemaphoreType.DMA,
            pltpu.SemaphoreType.DMA,
        ],
    )
    def body(fwd_hbm, xpad_hbm, out_hbm, fwd_v, idxb, buf0, buf1,
             outb0, outb1, sem0, sem1, osem0, osem1):
        wid = lax.axis_index("s") * nc + lax.axis_index("c")
        base = wid * npw
        pltpu.sync_copy(fwd_hbm, fwd_v)
        lanes = jnp.arange(16, dtype=jnp.int32)
        for gg in range(npw // 16):
            i_vec = base + gg * 16 + lanes
            loc = gg * 16 + lanes
            for k in range(KP1):
                nk = plsc.load_gather(fwd_v, [k * N + i_vec])
                m = plsc.load_gather(fwd_v, [nk]) == i_vec
                for j in range(1, KP1):
                    m = m | (plsc.load_gather(fwd_v, [j * N + nk]) == i_vec)
                eff = jnp.where(m, nk, jnp.int32(ZERO_ROW))
                plsc.store_scatter(
                    idxb,
                    [lax.shift_right_logical(loc, 2),
                     (loc & (GN - 1)) * KP1 + k],
                    eff)
        bufs = (buf0, buf1)
        sems = (sem0, sem1)
        outbs = (outb0, outb1)
        osems = (osem0, osem1)
        pltpu.async_copy(xpad_hbm.at[idxb.at[0]], buf0, sem0)
        for g in range(ng):
            p = g % 2
            cur = bufs[p]
            ob = outbs[p]
            if g + 1 < ng:
                pltpu.async_copy(xpad_hbm.at[idxb.at[g + 1]],
                                 bufs[1 - p], sems[1 - p])
            pltpu.make_async_copy(xpad_hbm.at[idxb.at[g]], cur,
                                  sems[p]).wait()
            if g >= 2:
                pltpu.make_async_copy(
                    ob, out_hbm.at[pl.ds(base + (g - 2) * GN, GN)],
                    osems[p]).wait()

            @plsc.parallel_loop(0, D // 16, unroll=4)
            def _vbody(v):
                sl = pl.ds(v * 16, 16)
                for n in range(GN):
                    s = cur[n * KP1, sl]
                    for k in range(1, KP1):
                        s = s + cur[n * KP1 + k, sl]
                    ob[n, sl] = s

            pltpu.async_copy(ob, out_hbm.at[pl.ds(base + g * GN, GN)],
                             osems[p])
        for g in (ng - 2, ng - 1):
            pltpu.make_async_copy(
                outbs[g % 2], out_hbm.at[pl.ds(base + g * GN, GN)],
                osems[g % 2]).wait()

    return body(fwd_flat, xpad)


def _mlp1_body(x_ref, ns_ref, w_ref, b_ref, o_ref):
    hin = (1.0 + EPS_GIN) * x_ref[...] + ns_ref[...]
    acc = lax.dot_general(hin, w_ref[...], (((1,), (1,)), ((), ())),
                          preferred_element_type=jnp.float32)
    o_ref[...] = jnp.maximum(acc + b_ref[0:1, :], 0.0)


def _mlp1(x, nsum, w, b8):
    return pl.pallas_call(
        _mlp1_body,
        grid=(N // BR,),
        in_specs=[
            pl.BlockSpec((BR, D), lambda i: (i, 0)),
            pl.BlockSpec((BR, D), lambda i: (i, 0)),
            pl.BlockSpec((D, D), lambda i: (0, 0)),
            pl.BlockSpec((8, D), lambda i: (0, 0)),
        ],
        out_specs=pl.BlockSpec((BR, D), lambda i: (i, 0)),
        out_shape=jax.ShapeDtypeStruct((N, D), jnp.float32),
    )(x, nsum, w, b8)


def _mlp2_body(h_ref, w_ref, b_ref, o_ref, cs_ref, cq_ref):
    i = pl.program_id(0)
    acc = lax.dot_general(h_ref[...], w_ref[...], (((1,), (1,)), ((), ())),
                          preferred_element_type=jnp.float32)
    acc = acc + b_ref[0:1, :]
    o_ref[...] = acc
    psum = jnp.sum(acc, axis=0, keepdims=True)
    psq = jnp.sum(acc * acc, axis=0, keepdims=True)

    @pl.when(i == 0)
    def _():
        cs_ref[...] = jnp.broadcast_to(psum, cs_ref.shape)
        cq_ref[...] = jnp.broadcast_to(psq, cq_ref.shape)

    @pl.when(i > 0)
    def _():
        cs_ref[0:1, :] += psum
        cq_ref[0:1, :] += psq


def _mlp2(h, w, b8):
    return pl.pallas_call(
        _mlp2_body,
        grid=(N // BR,),
        in_specs=[
            pl.BlockSpec((BR, D), lambda i: (i, 0)),
            pl.BlockSpec((D, D), lambda i: (0, 0)),
            pl.BlockSpec((8, D), lambda i: (0, 0)),
        ],
        out_specs=[
            pl.BlockSpec((BR, D), lambda i: (i, 0)),
            pl.BlockSpec((8, D), lambda i: (0, 0)),
            pl.BlockSpec((8, D), lambda i: (0, 0)),
        ],
        out_shape=[
            jax.ShapeDtypeStruct((N, D), jnp.float32),
            jax.ShapeDtypeStruct((8, D), jnp.float32),
            jax.ShapeDtypeStruct((8, D), jnp.float32),
        ],
    )(h, w, b8)


def _cls_body(g_ref, cs_ref, cq_ref, gam_ref, bet_ref, wc_ref, o_ref):
    mu = cs_ref[0:1, :] * (1.0 / N)
    var = cq_ref[0:1, :] * (1.0 / N) - mu * mu
    scale = gam_ref[0:1, :] / jnp.sqrt(var + 1e-5)
    shift = bet_ref[0:1, :] - mu * scale
    z = g_ref[...] * scale + shift
    o_ref[...] = lax.dot_general(z, wc_ref[...], (((1,), (1,)), ((), ())),
                                 preferred_element_type=jnp.float32)


def _cls(g, cs8, cq8, gam8, bet8, wc):
    return pl.pallas_call(
        _cls_body,
        grid=(N // BR,),
        in_specs=[
            pl.BlockSpec((BR, D), lambda i: (i, 0)),
            pl.BlockSpec((8, D), lambda i: (0, 0)),
            pl.BlockSpec((8, D), lambda i: (0, 0)),
            pl.BlockSpec((8, D), lambda i: (0, 0)),
            pl.BlockSpec((8, D), lambda i: (0, 0)),
            pl.BlockSpec((NUM_CLASSES, D), lambda i: (0, 0)),
        ],
        out_specs=pl.BlockSpec((BR, NUM_CLASSES), lambda i: (i, 0)),
        out_shape=jax.ShapeDtypeStruct((N, NUM_CLASSES), jnp.float32),
    )(g, cs8, cq8, gam8, bet8, wc)


def kernel(x, W1, b1, W2, b2, gamma, beta, Wc):
    rn8 = _norms(x)
    fwd8 = _topk(x, rn8)
    xpad = jnp.concatenate([x, jnp.zeros((8, D), x.dtype)], axis=0)
    nsum = _sc_neigh_sum(fwd8[:KP1].reshape(-1), xpad)
    b1g = jnp.broadcast_to(b1[None, :], (8, D))
    b2g = jnp.broadcast_to(b2[None, :], (8, D))
    gam8 = jnp.broadcast_to(gamma[None, :], (8, D))
    bet8 = jnp.broadcast_to(beta[None, :], (8, D))
    h1 = _mlp1(x, nsum, W1, b1g)
    g, cs8, cq8 = _mlp2(h1, W2, b2g)
    return _cls(g, cs8, cq8, gam8, bet8, Wc)


# speedup vs baseline: 1.1686x; 1.1686x over previous
import functools

import jax
import jax.numpy as jnp
from jax import lax
from jax.experimental import pallas as pl
from jax.experimental.pallas import tpu as pltpu
from jax.experimental.pallas import tpu_sc as plsc

N = 2048
D = 2048
NUM_CLASSES = 751
KP1 = 6
EPS_GIN = 0.3
ZERO_ROW = N

BR = 256


def _norm_body(x_ref, o_ref):
    x = x_ref[...]
    n = jnp.sqrt(jnp.sum(x * x, axis=1, keepdims=True))
    o_ref[...] = x / (n + 1e-12)


def _normalize(x):
    return pl.pallas_call(
        _norm_body,
        grid=(N // BR,),
        in_specs=[pl.BlockSpec((BR, D), lambda i: (i, 0))],
        out_specs=pl.BlockSpec((BR, D), lambda i: (i, 0)),
        out_shape=jax.ShapeDtypeStruct((N, D), jnp.float32),
    )(x)


def _topk_body(an_blk_ref, an_full_ref, idx_ref):
    a = an_blk_ref[...]
    b = an_full_ref[...]
    sim = lax.dot_general(a, b, (((1,), (1,)), ((), ())),
                          preferred_element_type=jnp.float32)
    d = 1.0 - sim
    col = lax.broadcasted_iota(jnp.int32, d.shape, 1)
    for t in range(KP1):
        m = jnp.min(d, axis=1, keepdims=True)
        idx = jnp.min(jnp.where(d == m, col, N), axis=1)
        idx_ref[t, :] = idx
        d = jnp.where(col == idx[:, None], jnp.float32(1e9), d)
    zero = jnp.zeros((idx_ref.shape[1],), jnp.int32)
    for t in range(KP1, 8):
        idx_ref[t, :] = zero


def _topk(an):
    return pl.pallas_call(
        _topk_body,
        grid=(N // BR,),
        in_specs=[
            pl.BlockSpec((BR, D), lambda i: (i, 0)),
            pl.BlockSpec((N, D), lambda i: (0, 0)),
        ],
        out_specs=pl.BlockSpec((8, BR), lambda i: (0, i)),
        out_shape=jax.ShapeDtypeStruct((8, N), jnp.int32),
    )(an, an)


GN = 4
NG = None


def _sc_neigh_sum(fwd_flat, xpad):
    info = plsc.get_sparse_core_info()
    nc, ns = info.num_cores, info.num_subcores
    nw = nc * ns
    npw = N // nw
    ng = npw // GN
    rows = GN * KP1
    mesh = plsc.VectorSubcoreMesh(core_axis_name="c", subcore_axis_name="s")

    @functools.partial(
        pl.kernel,
        out_type=jax.ShapeDtypeStruct((N, D), jnp.float32),
        mesh=mesh,
        compiler_params=pltpu.CompilerParams(needs_layout_passes=False),
        scratch_types=[
            pltpu.VMEM((KP1 * N,), jnp.int32),
            pltpu.VMEM((ng, rows), jnp.int32),
            pltpu.VMEM((rows, D), jnp.float32),
            pltpu.VMEM((rows, D), jnp.float32),
            pltpu.VMEM((GN, D), jnp.float32),
            pltpu.VMEM((GN, D), jnp.float32),
            pltpu.SemaphoreType.DMA,
            pltpu.SemaphoreType.DMA,
            pltpu.SemaphoreType.DMA,
            pltpu.SemaphoreType.DMA,
        ],
    )
    def body(fwd_hbm, xpad_hbm, out_hbm, fwd_v, idxb, buf0, buf1,
             outb0, outb1, sem0, sem1, osem0, osem1):
        wid = lax.axis_index("s") * nc + lax.axis_index("c")
        base = wid * npw
        pltpu.sync_copy(fwd_hbm, fwd_v)
        lanes = jnp.arange(16, dtype=jnp.int32)
        for gg in range(npw // 16):
            i_vec = base + gg * 16 + lanes
            loc = gg * 16 + lanes
            for k in range(KP1):
                nk = plsc.load_gather(fwd_v, [k * N + i_vec])
                m = plsc.load_gather(fwd_v, [nk]) == i_vec
                for j in range(1, KP1):
                    m = m | (plsc.load_gather(fwd_v, [j * N + nk]) == i_vec)
                eff = jnp.where(m, nk, jnp.int32(ZERO_ROW))
                plsc.store_scatter(
                    idxb,
                    [lax.shift_right_logical(loc, 2),
                     (loc & (GN - 1)) * KP1 + k],
                    eff)
        bufs = (buf0, buf1)
        sems = (sem0, sem1)
        outbs = (outb0, outb1)
        osems = (osem0, osem1)
        pltpu.async_copy(xpad_hbm.at[idxb.at[0]], buf0, sem0)
        for g in range(ng):
            p = g % 2
            cur = bufs[p]
            ob = outbs[p]
            if g + 1 < ng:
                pltpu.async_copy(xpad_hbm.at[idxb.at[g + 1]],
                                 bufs[1 - p], sems[1 - p])
            pltpu.make_async_copy(xpad_hbm.at[idxb.at[g]], cur,
                                  sems[p]).wait()
            if g >= 2:
                pltpu.make_async_copy(
                    ob, out_hbm.at[pl.ds(base + (g - 2) * GN, GN)],
                    osems[p]).wait()

            @plsc.parallel_loop(0, D // 16, unroll=4)
            def _vbody(v):
                sl = pl.ds(v * 16, 16)
                for n in range(GN):
                    s = cur[n * KP1, sl]
                    for k in range(1, KP1):
                        s = s + cur[n * KP1 + k, sl]
                    ob[n, sl] = s

            pltpu.async_copy(ob, out_hbm.at[pl.ds(base + g * GN, GN)],
                             osems[p])
        for g in (ng - 2, ng - 1):
            pltpu.make_async_copy(
                outbs[g % 2], out_hbm.at[pl.ds(base + g * GN, GN)],
                osems[g % 2]).wait()

    return body(fwd_flat, xpad)


def _mlp1_body(x_ref, ns_ref, w_ref, b_ref, o_ref):
    hin = (1.0 + EPS_GIN) * x_ref[...] + ns_ref[...]
    acc = lax.dot_general(hin, w_ref[...], (((1,), (1,)), ((), ())),
                          preferred_element_type=jnp.float32)
    o_ref[...] = jnp.maximum(acc + b_ref[0:1, :], 0.0)


def _mlp1(x, nsum, w, b8):
    return pl.pallas_call(
        _mlp1_body,
        grid=(N // BR,),
        in_specs=[
            pl.BlockSpec((BR, D), lambda i: (i, 0)),
            pl.BlockSpec((BR, D), lambda i: (i, 0)),
            pl.BlockSpec((D, D), lambda i: (0, 0)),
            pl.BlockSpec((8, D), lambda i: (0, 0)),
        ],
        out_specs=pl.BlockSpec((BR, D), lambda i: (i, 0)),
        out_shape=jax.ShapeDtypeStruct((N, D), jnp.float32),
    )(x, nsum, w, b8)


def _mlp2_body(h_ref, w_ref, b_ref, o_ref, cs_ref, cq_ref):
    i = pl.program_id(0)
    acc = lax.dot_general(h_ref[...], w_ref[...], (((1,), (1,)), ((), ())),
                          preferred_element_type=jnp.float32)
    acc = acc + b_ref[0:1, :]
    o_ref[...] = acc
    psum = jnp.sum(acc, axis=0, keepdims=True)
    psq = jnp.sum(acc * acc, axis=0, keepdims=True)

    @pl.when(i == 0)
    def _():
        cs_ref[...] = jnp.broadcast_to(psum, cs_ref.shape)
        cq_ref[...] = jnp.broadcast_to(psq, cq_ref.shape)

    @pl.when(i > 0)
    def _():
        cs_ref[0:1, :] += psum
        cq_ref[0:1, :] += psq


def _mlp2(h, w, b8):
    return pl.pallas_call(
        _mlp2_body,
        grid=(N // BR,),
        in_specs=[
            pl.BlockSpec((BR, D), lambda i: (i, 0)),
            pl.BlockSpec((D, D), lambda i: (0, 0)),
            pl.BlockSpec((8, D), lambda i: (0, 0)),
        ],
        out_specs=[
            pl.BlockSpec((BR, D), lambda i: (i, 0)),
            pl.BlockSpec((8, D), lambda i: (0, 0)),
            pl.BlockSpec((8, D), lambda i: (0, 0)),
        ],
        out_shape=[
            jax.ShapeDtypeStruct((N, D), jnp.float32),
            jax.ShapeDtypeStruct((8, D), jnp.float32),
            jax.ShapeDtypeStruct((8, D), jnp.float32),
        ],
    )(h, w, b8)


def _cls_body(g_ref, cs_ref, cq_ref, gam_ref, bet_ref, wc_ref, o_ref):
    mu = cs_ref[0:1, :] * (1.0 / N)
    var = cq_ref[0:1, :] * (1.0 / N) - mu * mu
    scale = gam_ref[0:1, :] / jnp.sqrt(var + 1e-5)
    shift = bet_ref[0:1, :] - mu * scale
    z = g_ref[...] * scale + shift
    o_ref[...] = lax.dot_general(z, wc_ref[...], (((1,), (1,)), ((), ())),
                                 preferred_element_type=jnp.float32)


def _cls(g, cs8, cq8, gam8, bet8, wc):
    return pl.pallas_call(
        _cls_body,
        grid=(N // BR,),
        in_specs=[
            pl.BlockSpec((BR, D), lambda i: (i, 0)),
            pl.BlockSpec((8, D), lambda i: (0, 0)),
            pl.BlockSpec((8, D), lambda i: (0, 0)),
            pl.BlockSpec((8, D), lambda i: (0, 0)),
            pl.BlockSpec((8, D), lambda i: (0, 0)),
            pl.BlockSpec((NUM_CLASSES, D), lambda i: (0, 0)),
        ],
        out_specs=pl.BlockSpec((BR, NUM_CLASSES), lambda i: (i, 0)),
        out_shape=jax.ShapeDtypeStruct((N, NUM_CLASSES), jnp.float32),
    )(g, cs8, cq8, gam8, bet8, wc)


def kernel(x, W1, b1, W2, b2, gamma, beta, Wc):
    an = _normalize(x)
    fwd8 = _topk(an)
    xpad = jnp.concatenate([x, jnp.zeros((8, D), x.dtype)], axis=0)
    nsum = _sc_neigh_sum(fwd8[:KP1].reshape(-1), xpad)
    b1g = jnp.broadcast_to(b1[None, :], (8, D))
    b2g = jnp.broadcast_to(b2[None, :], (8, D))
    gam8 = jnp.broadcast_to(gamma[None, :], (8, D))
    bet8 = jnp.broadcast_to(beta[None, :], (8, D))
    h1 = _mlp1(x, nsum, W1, b1g)
    g, cs8, cq8 = _mlp2(h1, W2, b2g)
    return _cls(g, cs8, cq8, gam8, bet8, Wc)
